# baseline (device time: 92241 ns/iter reference)
import functools

import jax
import jax.numpy as jnp
from jax import lax
from jax.experimental import pallas as pl
from jax.experimental.pallas import tpu as pltpu

N_DEV = 4
SQ = 1024
ROWS = SQ // N_DEV
HALF = ROWS // 2
SKV_SH = 1024
HQ = 8
DH = 128
D = HQ * DH
BLK = 64
SCALE = 0.08838834764831843


def kernel(x, Wq, K_ext, V_ext, Wo):
    def body(x_ref, wq_ref, k_ref, v_ref, wo_ref, out_ref,
             q_ref, k_bf, v_bf, wo_bf, bias_ref,
             ctx_parts, l_parts, comm_ctx, comm_l,
             ctx_send_sems, ctx_recv_sems, l_send_sems, l_recv_sems,
             out_send_sems, out_recv_sems):
        my = lax.axis_index("i")

        barrier = pltpu.get_barrier_semaphore()
        for o in (1, 2, 3):
            pl.semaphore_signal(
                barrier, inc=1,
                device_id=((my + o) % N_DEV,),
                device_id_type=pl.DeviceIdType.MESH,
            )
        pl.semaphore_wait(barrier, 3)

        k_bf[...] = k_ref[0].astype(jnp.bfloat16)
        v_bf[...] = v_ref[0].astype(jnp.bfloat16)
        wo_bf[...] = wo_ref[...].astype(jnp.bfloat16)

        q_ref[...] = (jnp.dot(
            x_ref[0].astype(jnp.bfloat16), wq_ref[...].astype(jnp.bfloat16),
            preferred_element_type=jnp.float32) * SCALE).astype(jnp.bfloat16)

        qb = lax.broadcasted_iota(jnp.int32, (SQ, SKV_SH), 0) // BLK
        kb = (lax.broadcasted_iota(jnp.int32, (SQ, SKV_SH), 1)
              + my * SKV_SH) // BLK
        mask = (qb == kb) | (kb == 0) | ((qb + kb) % 3 == 0)
        bias_ref[...] = jnp.where(
            mask, jnp.float32(0), jnp.float32(-1e9)).astype(jnp.bfloat16)

        rs_rdmas = []
        ones_bf = jnp.ones((SKV_SH, HQ), jnp.bfloat16)
        for o in (1, 2, 3, 0):
            owner = (my + o) % N_DEV
            row0 = owner * ROWS
            bias = bias_ref[pl.ds(row0, ROWS), :]

            for h in range(HQ):
                q_h = q_ref[pl.ds(row0, ROWS), pl.ds(h * DH, DH)]
                s = lax.dot_general(
                    q_h, k_bf[:, h, :], (((1,), (1,)), ((), ())),
                    preferred_element_type=jnp.float32,
                )
                p = jnp.exp(s.astype(jnp.bfloat16) + bias)
                l_parts[o, :, h * HQ:(h + 1) * HQ] = jnp.dot(
                    p, ones_bf, preferred_element_type=jnp.float32)
                ctx_parts[o, :, h * DH:(h + 1) * DH] = jnp.dot(
                    p, v_bf[:, h, :],
                    preferred_element_type=jnp.float32).astype(jnp.bfloat16)

            if o != 0:
                slot = 3 - o
                rc = pltpu.make_async_remote_copy(
                    src_ref=ctx_parts.at[o],
                    dst_ref=comm_ctx.at[slot],
                    send_sem=ctx_send_sems.at[o - 1],
                    recv_sem=ctx_recv_sems.at[slot],
                    device_id=(owner,),
                    device_id_type=pl.DeviceIdType.MESH,
                )
                rl = pltpu.make_async_remote_copy(
                    src_ref=l_parts.at[o],
                    dst_ref=comm_l.at[slot],
                    send_sem=l_send_sems.at[o - 1],
                    recv_sem=l_recv_sems.at[slot],
                    device_id=(owner,),
                    device_id_type=pl.DeviceIdType.MESH,
                )
                rc.start()
                rl.start()
                rs_rdmas.append((rc, rl))

        for rc, rl in rs_rdmas:
            rc.wait_recv()
            rl.wait_recv()

        ctx_mine = (ctx_parts[0].astype(jnp.float32)
                    + comm_ctx[0].astype(jnp.float32)
                    + comm_ctx[1].astype(jnp.float32)
                    + comm_ctx[2].astype(jnp.float32))
        l_mine = l_parts[0] + comm_l[0] + comm_l[1] + comm_l[2]
        cols = []
        for h in range(HQ):
            denom = l_mine[:, h * HQ:h * HQ + 1]
            cols.append(ctx_mine[:, h * DH:(h + 1) * DH] / denom)
        attn = jnp.concatenate(cols, axis=1)

        ag_rdmas = []
        for half in range(2):
            r0 = my * ROWS + half * HALF
            out_ref[0, pl.ds(r0, HALF), :] = jnp.dot(
                attn[half * HALF:(half + 1) * HALF, :].astype(jnp.bfloat16),
                wo_bf[...], preferred_element_type=jnp.float32)
            for o in (1, 2, 3):
                peer = (my + o) % N_DEV
                ro = pltpu.make_async_remote_copy(
                    src_ref=out_ref.at[0, pl.ds(r0, HALF), :],
                    dst_ref=out_ref.at[0, pl.ds(r0, HALF), :],
                    send_sem=out_send_sems.at[(o - 1) * 2 + half],
                    recv_sem=out_recv_sems.at[(3 - o) * 2 + half],
                    device_id=(peer,),
                    device_id_type=pl.DeviceIdType.MESH,
                )
                ro.start()
                ag_rdmas.append(ro)

        for j in range(3):
            sender = (my + j + 1) % N_DEV
            for half in range(2):
                rw = pltpu.make_async_remote_copy(
                    src_ref=out_ref.at[0, pl.ds(my * ROWS, HALF), :],
                    dst_ref=out_ref.at[
                        0, pl.ds(sender * ROWS + half * HALF, HALF), :],
                    send_sem=out_send_sems.at[j * 2 + half],
                    recv_sem=out_recv_sems.at[j * 2 + half],
                    device_id=(sender,),
                    device_id_type=pl.DeviceIdType.MESH,
                )
                rw.wait_recv()

        for rc, rl in rs_rdmas:
            rc.wait_send()
            rl.wait_send()
        for ro in ag_rdmas:
            ro.wait_send()

        @functools.partial(pl.run_scoped, sem=pltpu.SemaphoreType.REGULAR)
        def _(sem):
            for o in (1, 2, 3):
                pl.semaphore_signal(
                    sem, inc=1,
                    device_id=((my + o) % N_DEV,),
                    device_id_type=pl.DeviceIdType.MESH,
                )
            pl.semaphore_wait(sem, 3)

    return pl.pallas_call(
        body,
        out_shape=jax.ShapeDtypeStruct((1, SQ, D), jnp.float32),
        in_specs=[pl.BlockSpec(memory_space=pltpu.VMEM)] * 5,
        out_specs=pl.BlockSpec(memory_space=pltpu.VMEM),
        scratch_shapes=[
            pltpu.VMEM((SQ, D), jnp.bfloat16),
            pltpu.VMEM((SKV_SH, HQ, DH), jnp.bfloat16),
            pltpu.VMEM((SKV_SH, HQ, DH), jnp.bfloat16),
            pltpu.VMEM((D, D), jnp.bfloat16),
            pltpu.VMEM((SQ, SKV_SH), jnp.bfloat16),
            pltpu.VMEM((N_DEV, ROWS, D), jnp.bfloat16),
            pltpu.VMEM((N_DEV, ROWS, HQ * HQ), jnp.float32),
            pltpu.VMEM((3, ROWS, D), jnp.bfloat16),
            pltpu.VMEM((3, ROWS, HQ * HQ), jnp.float32),
            pltpu.SemaphoreType.DMA((3,)),
            pltpu.SemaphoreType.DMA((3,)),
            pltpu.SemaphoreType.DMA((3,)),
            pltpu.SemaphoreType.DMA((3,)),
            pltpu.SemaphoreType.DMA((6,)),
            pltpu.SemaphoreType.DMA((6,)),
        ],
        compiler_params=pltpu.CompilerParams(collective_id=0),
    )(x, Wq, K_ext, V_ext, Wo)


# device time: 78166 ns/iter; 1.1801x vs baseline; 1.1801x over previous
import functools

import jax
import jax.numpy as jnp
from jax import lax
from jax.experimental import pallas as pl
from jax.experimental.pallas import tpu as pltpu

N_DEV = 4
SQ = 1024
ROWS = SQ // N_DEV
HALF = ROWS // 2
SKV_SH = 1024
HQ = 8
DH = 128
D = HQ * DH
BLK = 64
SCALE = 0.08838834764831843


def kernel(x, Wq, K_ext, V_ext, Wo):
    def body(x_ref, wq_ref, k_ref, v_ref, wo_ref, out_ref,
             q_ref, kT_bf, v_bf, wo_bf, bias_ref, out_bf,
             ctx_parts, l_parts, comm_ctx, comm_l,
             ctx_send_sems, ctx_recv_sems, l_send_sems, l_recv_sems,
             out_send_sems, out_recv_sems):
        my = lax.axis_index("i")

        barrier = pltpu.get_barrier_semaphore()
        for o in (1, 2, 3):
            pl.semaphore_signal(
                barrier, inc=1,
                device_id=((my + o) % N_DEV,),
                device_id_type=pl.DeviceIdType.MESH,
            )
        pl.semaphore_wait(barrier, 3)

        for h in range(HQ):
            kT_bf[h, :, :] = k_ref[0, :, h, :].astype(jnp.bfloat16).T
        v_bf[...] = v_ref[0].astype(jnp.bfloat16)
        wo_bf[...] = wo_ref[...].astype(jnp.bfloat16)

        q_ref[...] = (jnp.dot(
            x_ref[0].astype(jnp.bfloat16), wq_ref[...].astype(jnp.bfloat16),
            preferred_element_type=jnp.float32) * SCALE).astype(jnp.bfloat16)

        qb = lax.broadcasted_iota(jnp.int32, (SQ, SKV_SH), 0) // BLK
        kb = (lax.broadcasted_iota(jnp.int32, (SQ, SKV_SH), 1)
              + my * SKV_SH) // BLK
        mask = (qb == kb) | (kb == 0) | ((qb + kb) % 3 == 0)
        bias_ref[...] = jnp.where(
            mask, jnp.float32(0), jnp.float32(-1e9)).astype(jnp.bfloat16)

        rs_rdmas = []
        ones_bf = jnp.ones((SKV_SH, HQ), jnp.bfloat16)
        for o in (1, 2, 3, 0):
            owner = (my + o) % N_DEV
            row0 = owner * ROWS
            bias = bias_ref[pl.ds(row0, ROWS), :]

            for h in range(HQ):
                q_h = q_ref[pl.ds(row0, ROWS), pl.ds(h * DH, DH)]
                s = jnp.dot(q_h, kT_bf[h], preferred_element_type=jnp.float32)
                p = jnp.exp(s.astype(jnp.bfloat16) + bias)
                l_parts[o, :, h * HQ:(h + 1) * HQ] = jnp.dot(
                    p, ones_bf, preferred_element_type=jnp.float32)
                ctx_parts[o, :, h * DH:(h + 1) * DH] = jnp.dot(
                    p, v_bf[:, h, :],
                    preferred_element_type=jnp.float32).astype(jnp.bfloat16)

            if o != 0:
                slot = 3 - o
                rc = pltpu.make_async_remote_copy(
                    src_ref=ctx_parts.at[o],
                    dst_ref=comm_ctx.at[slot],
                    send_sem=ctx_send_sems.at[o - 1],
                    recv_sem=ctx_recv_sems.at[slot],
                    device_id=(owner,),
                    device_id_type=pl.DeviceIdType.MESH,
                )
                rl = pltpu.make_async_remote_copy(
                    src_ref=l_parts.at[o],
                    dst_ref=comm_l.at[slot],
                    send_sem=l_send_sems.at[o - 1],
                    recv_sem=l_recv_sems.at[slot],
                    device_id=(owner,),
                    device_id_type=pl.DeviceIdType.MESH,
                )
                rc.start()
                rl.start()
                rs_rdmas.append((rc, rl))

        for rc, rl in rs_rdmas:
            rc.wait_recv()
            rl.wait_recv()

        ctx_mine = (ctx_parts[0].astype(jnp.float32)
                    + comm_ctx[0].astype(jnp.float32)
                    + comm_ctx[1].astype(jnp.float32)
                    + comm_ctx[2].astype(jnp.float32))
        l_mine = l_parts[0] + comm_l[0] + comm_l[1] + comm_l[2]
        cols = []
        for h in range(HQ):
            denom = l_mine[:, h * HQ:h * HQ + 1]
            cols.append(ctx_mine[:, h * DH:(h + 1) * DH] / denom)
        attn = jnp.concatenate(cols, axis=1)

        ag_rdmas = []
        for half in range(2):
            r0 = my * ROWS + half * HALF
            out_bf[pl.ds(r0, HALF), :] = jnp.dot(
                attn[half * HALF:(half + 1) * HALF, :].astype(jnp.bfloat16),
                wo_bf[...], preferred_element_type=jnp.float32
            ).astype(jnp.bfloat16)
            for o in (1, 2, 3):
                peer = (my + o) % N_DEV
                ro = pltpu.make_async_remote_copy(
                    src_ref=out_bf.at[pl.ds(r0, HALF), :],
                    dst_ref=out_bf.at[pl.ds(r0, HALF), :],
                    send_sem=out_send_sems.at[(o - 1) * 2 + half],
                    recv_sem=out_recv_sems.at[(3 - o) * 2 + half],
                    device_id=(peer,),
                    device_id_type=pl.DeviceIdType.MESH,
                )
                ro.start()
                ag_rdmas.append(ro)

        for j in range(3):
            sender = (my + j + 1) % N_DEV
            for half in range(2):
                rw = pltpu.make_async_remote_copy(
                    src_ref=out_bf.at[pl.ds(my * ROWS, HALF), :],
                    dst_ref=out_bf.at[
                        pl.ds(sender * ROWS + half * HALF, HALF), :],
                    send_sem=out_send_sems.at[j * 2 + half],
                    recv_sem=out_recv_sems.at[j * 2 + half],
                    device_id=(sender,),
                    device_id_type=pl.DeviceIdType.MESH,
                )
                rw.wait_recv()

        out_ref[0] = out_bf[...].astype(jnp.float32)

        for rc, rl in rs_rdmas:
            rc.wait_send()
            rl.wait_send()
        for ro in ag_rdmas:
            ro.wait_send()

        @functools.partial(pl.run_scoped, sem=pltpu.SemaphoreType.REGULAR)
        def _(sem):
            for o in (1, 2, 3):
                pl.semaphore_signal(
                    sem, inc=1,
                    device_id=((my + o) % N_DEV,),
                    device_id_type=pl.DeviceIdType.MESH,
                )
            pl.semaphore_wait(sem, 3)

    return pl.pallas_call(
        body,
        out_shape=jax.ShapeDtypeStruct((1, SQ, D), jnp.float32),
        in_specs=[pl.BlockSpec(memory_space=pltpu.VMEM)] * 5,
        out_specs=pl.BlockSpec(memory_space=pltpu.VMEM),
        scratch_shapes=[
            pltpu.VMEM((SQ, D), jnp.bfloat16),
            pltpu.VMEM((HQ, DH, SKV_SH), jnp.bfloat16),
            pltpu.VMEM((SKV_SH, HQ, DH), jnp.bfloat16),
            pltpu.VMEM((D, D), jnp.bfloat16),
            pltpu.VMEM((SQ, SKV_SH), jnp.bfloat16),
            pltpu.VMEM((SQ, D), jnp.bfloat16),
            pltpu.VMEM((N_DEV, ROWS, D), jnp.bfloat16),
            pltpu.VMEM((N_DEV, ROWS, HQ * HQ), jnp.float32),
            pltpu.VMEM((3, ROWS, D), jnp.bfloat16),
            pltpu.VMEM((3, ROWS, HQ * HQ), jnp.float32),
            pltpu.SemaphoreType.DMA((3,)),
            pltpu.SemaphoreType.DMA((3,)),
            pltpu.SemaphoreType.DMA((3,)),
            pltpu.SemaphoreType.DMA((3,)),
            pltpu.SemaphoreType.DMA((6,)),
            pltpu.SemaphoreType.DMA((6,)),
        ],
        compiler_params=pltpu.CompilerParams(collective_id=0),
    )(x, Wq, K_ext, V_ext, Wo)


# device time: 71523 ns/iter; 1.2897x vs baseline; 1.0929x over previous
import functools

import jax
import jax.numpy as jnp
from jax import lax
from jax.experimental import pallas as pl
from jax.experimental.pallas import tpu as pltpu

N_DEV = 4
SQ = 1024
ROWS = SQ // N_DEV
HALF = ROWS // 2
SKV_SH = 1024
NQB = SQ // 64
HQ = 8
DH = 128
D = HQ * DH
BLK = 64
SCALE = 0.08838834764831843

NSEG = 6 * BLK
GROUP_START = (0, 384, 704)
GROUP_N = (384, 320, 320)


def _grouppos(qb):
    return GROUP_START[qb % 3] + BLK * (qb // 3)


def _segpos(jb):
    return NSEG * (jb % 3) + BLK * (jb // 3)


def kernel(x, Wq, K_ext, V_ext, Wo):
    def body(x_ref, wq_ref, k_ref, v_ref, wo_ref, out_ref,
             q_perm, kT_bf, kT_seg, v_perm, wo_bf,
             ctx_grp, l_grp, ctx_parts, l_parts, comm_ctx, comm_l, out_bf,
             ctx_send_sems, ctx_recv_sems, l_send_sems, l_recv_sems,
             out_send_sems, out_recv_sems):
        my = lax.axis_index("i")

        barrier = pltpu.get_barrier_semaphore()
        for o in (1, 2, 3):
            pl.semaphore_signal(
                barrier, inc=1,
                device_id=((my + o) % N_DEV,),
                device_id_type=pl.DeviceIdType.MESH,
            )
        pl.semaphore_wait(barrier, 3)

        for h in range(HQ):
            kT_bf[h, :, :] = k_ref[0, :, h, :].astype(jnp.bfloat16).T
        for jb in range(NQB):
            sp = _segpos(jb)
            kT_seg[:, :, sp:sp + BLK] = kT_bf[:, :, jb * BLK:(jb + 1) * BLK]
            v_perm[sp:sp + BLK] = v_ref[0, jb * BLK:(jb + 1) * BLK].astype(
                jnp.bfloat16)
        for pad0 in (NSEG + 5 * BLK, 2 * NSEG + 5 * BLK):
            kT_seg[:, :, pad0:pad0 + BLK] = jnp.zeros(
                (HQ, DH, BLK), jnp.bfloat16)
            v_perm[pad0:pad0 + BLK] = jnp.zeros((BLK, HQ, DH), jnp.bfloat16)
        wo_bf[...] = wo_ref[...].astype(jnp.bfloat16)

        qv = (jnp.dot(
            x_ref[0].astype(jnp.bfloat16), wq_ref[...].astype(jnp.bfloat16),
            preferred_element_type=jnp.float32) * SCALE).astype(jnp.bfloat16)
        for qb in range(NQB):
            q_perm[_grouppos(qb):_grouppos(qb) + BLK, :] = (
                qv[qb * BLK:(qb + 1) * BLK, :])

        ones_bf = jnp.ones((NSEG, HQ), jnp.bfloat16)

        for r in range(3):
            g0, gn = GROUP_START[r], GROUP_N[r]
            c = jnp.mod(-(my + r), 3)
            colblk = lax.broadcasted_iota(jnp.int32, (1, NSEG), 1) // BLK
            colbias = jnp.where(c + 3 * colblk < NQB, jnp.float32(0),
                                jnp.float32(-1e9)).astype(jnp.bfloat16)
            for h in range(HQ):
                q_h = q_perm[g0:g0 + gn, h * DH:(h + 1) * DH]
                s = jnp.dot(q_h, kT_seg[h, :, pl.ds(c * NSEG, NSEG)],
                            preferred_element_type=jnp.float32)
                p = jnp.exp(s.astype(jnp.bfloat16) + colbias)
                l_grp[g0:g0 + gn, h * HQ:(h + 1) * HQ] = jnp.dot(
                    p, ones_bf, preferred_element_type=jnp.float32)
                ctx_grp[g0:g0 + gn, h * DH:(h + 1) * DH] = jnp.dot(
                    p, v_perm[pl.ds(c * NSEG, NSEG), h, :],
                    preferred_element_type=jnp.float32).astype(jnp.bfloat16)

        @pl.when(my == 0)
        def _():
            for r in (1, 2):
                g0, gn = GROUP_START[r], GROUP_N[r]
                nblk = gn // BLK
                jbl = [0] + [r + 3 * t for t in range(nblk)]
                rowqb = r + 3 * (lax.broadcasted_iota(
                    jnp.int32, (gn, 6 * BLK), 0) // BLK)
                colj = lax.broadcasted_iota(
                    jnp.int32, (gn, 6 * BLK), 1) // BLK
                coljb = jnp.where(colj == 0, -1, r + 3 * (colj - 1))
                maskx = (colj == 0) | (rowqb == coljb)
                biasx = jnp.where(maskx, jnp.float32(0),
                                  jnp.float32(-1e9)).astype(jnp.bfloat16)
                for h in range(HQ):
                    kx = jnp.concatenate(
                        [kT_seg[h, :, _segpos(jb):_segpos(jb) + BLK]
                         for jb in jbl], axis=1)
                    vx = jnp.concatenate(
                        [v_perm[_segpos(jb):_segpos(jb) + BLK, h, :]
                         for jb in jbl], axis=0)
                    q_h = q_perm[g0:g0 + gn, h * DH:(h + 1) * DH]
                    sx = jnp.dot(q_h, kx, preferred_element_type=jnp.float32)
                    px = jnp.exp(sx.astype(jnp.bfloat16) + biasx)
                    l_grp[g0:g0 + gn, h * HQ:(h + 1) * HQ] = (
                        l_grp[g0:g0 + gn, h * HQ:(h + 1) * HQ]
                        + jnp.dot(px, ones_bf,
                                  preferred_element_type=jnp.float32))
                    ctx_grp[g0:g0 + gn, h * DH:(h + 1) * DH] = (
                        ctx_grp[g0:g0 + gn, h * DH:(h + 1) * DH]
                        .astype(jnp.float32)
                        + jnp.dot(px, vx, preferred_element_type=jnp.float32)
                    ).astype(jnp.bfloat16)

        rs_rdmas = []
        for o in (1, 2, 3, 0):
            owner = (my + o) % N_DEV
            for i in range(4):
                qb = 4 * owner + i
                g = jnp.mod(qb, 3)
                pos = (jnp.where(g == 0, 0, jnp.where(g == 1, 384, 704))
                       + BLK * (qb // 3))
                ctx_parts[o, i * BLK:(i + 1) * BLK, :] = (
                    ctx_grp[pl.ds(pos, BLK), :])
                l_parts[o, i * BLK:(i + 1) * BLK, :] = (
                    l_grp[pl.ds(pos, BLK), :])
            if o != 0:
                slot = 3 - o
                rc = pltpu.make_async_remote_copy(
                    src_ref=ctx_parts.at[o],
                    dst_ref=comm_ctx.at[slot],
                    send_sem=ctx_send_sems.at[o - 1],
                    recv_sem=ctx_recv_sems.at[slot],
                    device_id=(owner,),
                    device_id_type=pl.DeviceIdType.MESH,
                )
                rl = pltpu.make_async_remote_copy(
                    src_ref=l_parts.at[o],
                    dst_ref=comm_l.at[slot],
                    send_sem=l_send_sems.at[o - 1],
                    recv_sem=l_recv_sems.at[slot],
                    device_id=(owner,),
                    device_id_type=pl.DeviceIdType.MESH,
                )
                rc.start()
                rl.start()
                rs_rdmas.append((rc, rl))

        for rc, rl in rs_rdmas:
            rc.wait_recv()
            rl.wait_recv()

        ctx_mine = (ctx_parts[0].astype(jnp.float32)
                    + comm_ctx[0].astype(jnp.float32)
                    + comm_ctx[1].astype(jnp.float32)
                    + comm_ctx[2].astype(jnp.float32))
        l_mine = l_parts[0] + comm_l[0] + comm_l[1] + comm_l[2]
        cols = []
        for h in range(HQ):
            denom = l_mine[:, h * HQ:h * HQ + 1]
            cols.append(ctx_mine[:, h * DH:(h + 1) * DH] / denom)
        attn = jnp.concatenate(cols, axis=1)

        ag_rdmas = []
        for half in range(2):
            r0 = my * ROWS + half * HALF
            out_bf[pl.ds(r0, HALF), :] = jnp.dot(
                attn[half * HALF:(half + 1) * HALF, :].astype(jnp.bfloat16),
                wo_bf[...], preferred_element_type=jnp.float32
            ).astype(jnp.bfloat16)
            for o in (1, 2, 3):
                peer = (my + o) % N_DEV
                ro = pltpu.make_async_remote_copy(
                    src_ref=out_bf.at[pl.ds(r0, HALF), :],
                    dst_ref=out_bf.at[pl.ds(r0, HALF), :],
                    send_sem=out_send_sems.at[(o - 1) * 2 + half],
                    recv_sem=out_recv_sems.at[(3 - o) * 2 + half],
                    device_id=(peer,),
                    device_id_type=pl.DeviceIdType.MESH,
                )
                ro.start()
                ag_rdmas.append(ro)

        for j in range(3):
            sender = (my + j + 1) % N_DEV
            for half in range(2):
                rw = pltpu.make_async_remote_copy(
                    src_ref=out_bf.at[pl.ds(my * ROWS, HALF), :],
                    dst_ref=out_bf.at[
                        pl.ds(sender * ROWS + half * HALF, HALF), :],
                    send_sem=out_send_sems.at[j * 2 + half],
                    recv_sem=out_recv_sems.at[j * 2 + half],
                    device_id=(sender,),
                    device_id_type=pl.DeviceIdType.MESH,
                )
                rw.wait_recv()

        out_ref[0] = out_bf[...].astype(jnp.float32)

        for rc, rl in rs_rdmas:
            rc.wait_send()
            rl.wait_send()
        for ro in ag_rdmas:
            ro.wait_send()

        @functools.partial(pl.run_scoped, sem=pltpu.SemaphoreType.REGULAR)
        def _(sem):
            for o in (1, 2, 3):
                pl.semaphore_signal(
                    sem, inc=1,
                    device_id=((my + o) % N_DEV,),
                    device_id_type=pl.DeviceIdType.MESH,
                )
            pl.semaphore_wait(sem, 3)

    return pl.pallas_call(
        body,
        out_shape=jax.ShapeDtypeStruct((1, SQ, D), jnp.float32),
        in_specs=[pl.BlockSpec(memory_space=pltpu.VMEM)] * 5,
        out_specs=pl.BlockSpec(memory_space=pltpu.VMEM),
        scratch_shapes=[
            pltpu.VMEM((SQ, D), jnp.bfloat16),
            pltpu.VMEM((HQ, DH, SKV_SH), jnp.bfloat16),
            pltpu.VMEM((HQ, DH, 3 * NSEG), jnp.bfloat16),
            pltpu.VMEM((3 * NSEG, HQ, DH), jnp.bfloat16),
            pltpu.VMEM((D, D), jnp.bfloat16),
            pltpu.VMEM((SQ, D), jnp.bfloat16),
            pltpu.VMEM((SQ, HQ * HQ), jnp.float32),
            pltpu.VMEM((N_DEV, ROWS, D), jnp.bfloat16),
            pltpu.VMEM((N_DEV, ROWS, HQ * HQ), jnp.float32),
            pltpu.VMEM((3, ROWS, D), jnp.bfloat16),
            pltpu.VMEM((3, ROWS, HQ * HQ), jnp.float32),
            pltpu.VMEM((SQ, D), jnp.bfloat16),
            pltpu.SemaphoreType.DMA((3,)),
            pltpu.SemaphoreType.DMA((3,)),
            pltpu.SemaphoreType.DMA((3,)),
            pltpu.SemaphoreType.DMA((3,)),
            pltpu.SemaphoreType.DMA((6,)),
            pltpu.SemaphoreType.DMA((6,)),
        ],
        compiler_params=pltpu.CompilerParams(collective_id=0),
    )(x, Wq, K_ext, V_ext, Wo)
